# baseline (speedup 1.0000x reference)
import functools

import jax
import jax.numpy as jnp
from jax import lax
from jax.experimental import pallas as pl
from jax.experimental.pallas import tpu as pltpu
from jax.experimental.pallas import tpu_sc as plsc

N = 10000
E = 320000
D_IN = 128
D_HID = 256
D_OUT = 128

NC = 2
NS = 16
ROWS_PER_TILE = 640
NPAD = NS * ROWS_PER_TILE
CW = 128
SROWS = 16
EROWS = 2560
E_PAD = EROWS * CW
RB = 400


def _sc_mesh():
    return plsc.VectorSubcoreMesh(core_axis_name="c", subcore_axis_name="s")


HR = NPAD // 128
HRPT = 8
DEG_CHUNK = 2048


@functools.partial(
    pl.kernel,
    mesh=_sc_mesh(),
    compiler_params=pltpu.CompilerParams(needs_layout_passes=False),
    out_type=jax.ShapeDtypeStruct((NC, HR, 128), jnp.float32),
    scratch_types=[
        pltpu.VMEM((DEG_CHUNK,), jnp.int32),
        pltpu.VMEM((HR, 128), jnp.float32),
        pltpu.VMEM((HR,), jnp.int32),
        pltpu.VMEM((HRPT, 128), jnp.float32),
        pltpu.VMEM_SHARED((HR, 128), jnp.float32),
    ],
)
def _deg(dst_hbm, out_hbm, idx_v, hist_v, rowid_v, ob_v, acc_sh):
    c = lax.axis_index("c")
    s = lax.axis_index("s")

    def zrow(j, carry):
        def zcol(l, carry2):
            hist_v[j, pl.ds(l * 16, 16)] = jnp.zeros((16,), jnp.float32)
            return carry2
        return lax.fori_loop(0, 128 // 16, zcol, carry)

    lax.fori_loop(0, HR, zrow, 0)
    for j in range(HR // 16):
        rowid_v[pl.ds(j * 16, 16)] = (
            lax.iota(jnp.int32, 16) + jnp.full((16,), j * 16, jnp.int32))

    @pl.when(s == 0)
    def _():
        pltpu.sync_copy(hist_v, acc_sh)
    plsc.subcore_barrier()

    epw = E_PAD // (NC * NS)
    base = (c * NS + s) * epw

    def chunk_step(k, carry):
        pltpu.sync_copy(dst_hbm.at[pl.ds(base + k * DEG_CHUNK, DEG_CHUNK)],
                        idx_v)

        def vstep(j, carry2):
            v = idx_v[pl.ds(j * 16, 16)]
            skey, _ = plsc.sort_key_val(v, v)
            cnt, last = plsc.scan_count(skey)
            row = lax.shift_right_logical(skey, 7)
            col = jnp.bitwise_and(skey, 127)
            plsc.addupdate_scatter(hist_v, [row, col],
                                   cnt.astype(jnp.float32), mask=last)
            return carry2

        return lax.fori_loop(0, DEG_CHUNK // 16, vstep, carry)

    lax.fori_loop(0, epw // DEG_CHUNK, chunk_step, 0)
    pltpu.sync_copy(hist_v, acc_sh.at[rowid_v], add=True)
    plsc.subcore_barrier()

    @pl.when(s < HR // HRPT)
    def _():
        pltpu.sync_copy(acc_sh.at[pl.ds(s * HRPT, HRPT)], ob_v)

        @pl.when(c == 0)
        def _():
            pltpu.sync_copy(ob_v, out_hbm.at[0, pl.ds(s * HRPT, HRPT)])

        @pl.when(c == 1)
        def _():
            pltpu.sync_copy(ob_v, out_hbm.at[1, pl.ds(s * HRPT, HRPT)])


def _make_conv(F, feature_split, pipelined=True):
    zr = 64
    nz = ROWS_PER_TILE // zr
    rpt = EROWS // NS if feature_split else EROWS // (NC * NS)
    nstages = rpt // SROWS

    n_hs = 2 if feature_split else 1

    @functools.partial(
        pl.kernel,
        mesh=_sc_mesh(),
        out_type=[
            jax.ShapeDtypeStruct((NPAD, F), jnp.float32),
            jax.ShapeDtypeStruct((NPAD, F), jnp.float32),
        ],
        scratch_types=[
            pltpu.VMEM((SROWS, CW), jnp.int32),
            pltpu.VMEM((SROWS, CW), jnp.int32),
            pltpu.VMEM((CW, F), jnp.float32),
            pltpu.VMEM((CW, F), jnp.float32),
            pltpu.VMEM((zr, F), jnp.float32),
            pltpu.VMEM((80,), jnp.int32),
            pltpu.VMEM((80,), jnp.int32),
            pltpu.VMEM((80, F), jnp.float32),
            pltpu.SemaphoreType.DMA,
            pltpu.SemaphoreType.DMA,
            pltpu.VMEM_SHARED((NPAD, F), jnp.float32),
        ],
    )
    def conv(src_hbm, dst_hbm, *rest):
        hs_hbms = rest[:n_hs]
        out0_hbm, out1_hbm = rest[n_hs:n_hs + 2]
        (sidx, didx, rb0, rb1, zb_v, si80, di80, r80,
         sm0, sm1, acc_sh) = rest[n_hs + 2:]
        bufs = (rb0, rb1)
        sems = (sm0, sm1)
        NBUF = 2
        c = lax.axis_index("c")
        s = lax.axis_index("s")

        def zrow(j, carry):
            def zcol(l, carry2):
                zb_v[j, pl.ds(l * 16, 16)] = jnp.zeros((16,), jnp.float32)
                return carry2
            return lax.fori_loop(0, F // 16, zcol, carry)

        lax.fori_loop(0, zr, zrow, 0)
        row0 = s * ROWS_PER_TILE
        for t in range(nz):
            pltpu.sync_copy(zb_v, acc_sh.at[pl.ds(row0 + t * zr, zr)])
        plsc.subcore_barrier()

        def pass_edges(hs_hbm, first_stage, stride):
            def stage(oo, carry):
                r0 = (first_stage + oo * stride) * SROWS
                pltpu.sync_copy(src_hbm.at[pl.ds(r0, SROWS)], sidx)
                pltpu.sync_copy(dst_hbm.at[pl.ds(r0, SROWS)], didx)
                for u in range(NBUF - 1):
                    pltpu.async_copy(hs_hbm.at[sidx.at[u]], bufs[u], sems[u])

                def quad(q, c2):
                    j0 = q * NBUF
                    for u in range(NBUF):
                        j = j0 + u
                        nxt = j + NBUF - 1
                        bn = (u + NBUF - 1) % NBUF

                        @pl.when(nxt < SROWS)
                        def _():
                            pltpu.async_copy(
                                hs_hbm.at[sidx.at[nxt]], bufs[bn], sems[bn])

                        pltpu.make_async_copy(
                            hs_hbm.at[pl.ds(0, CW)], bufs[u], sems[u]).wait()
                        pltpu.sync_copy(
                            bufs[u], acc_sh.at[didx.at[j]], add=True)
                    return c2

                return lax.fori_loop(0, SROWS // NBUF, quad, carry)

            lax.fori_loop(0, nstages, stage, 0)

        if not pipelined:
            ept = E_PAD // (NC * NS)

            def step(i, carry):
                off = (c * NS + s) * ept + i * 80
                pltpu.sync_copy(src_hbm.at[pl.ds(off, 80)], si80)
                pltpu.sync_copy(dst_hbm.at[pl.ds(off, 80)], di80)
                pltpu.async_copy(hs_hbms[0].at[si80], r80, sems[0]).wait()
                pltpu.sync_copy(r80, acc_sh.at[di80], add=True)
                return carry

            lax.fori_loop(0, ept // 80, step, 0)
        elif feature_split:
            @pl.when(c == 0)
            def _():
                pass_edges(hs_hbms[0], s, NS)

            @pl.when(c == 1)
            def _():
                pass_edges(hs_hbms[1], s, NS)


        plsc.subcore_barrier()
        for t in range(nz):
            pltpu.sync_copy(acc_sh.at[pl.ds(row0 + t * zr, zr)], zb_v)

            @pl.when(c == 0)
            def _():
                pltpu.sync_copy(zb_v, out0_hbm.at[pl.ds(row0 + t * zr, zr)])

            @pl.when(c == 1)
            def _():
                pltpu.sync_copy(zb_v, out1_hbm.at[pl.ds(row0 + t * zr, zr)])

    return conv


_conv_hid = _make_conv(D_HID // 2, feature_split=True)
_conv_out = _make_conv(D_OUT, feature_split=False, pipelined=False)


def _mm1_body(degp_ref, x_ref, w_ref, hs0_ref, hs1_ref, dinv_ref):
    deg = degp_ref[:, 0] + degp_ref[:, 1] + 1.0
    dinv = lax.rsqrt(deg)
    h = jnp.dot(x_ref[...], w_ref[...], preferred_element_type=jnp.float32)
    hs = h * dinv[:, None]
    hs0_ref[...] = hs[:, : D_HID // 2]
    hs1_ref[...] = hs[:, D_HID // 2:]
    dinv_ref[...] = dinv[:, None]


_mm1 = pl.pallas_call(
    _mm1_body,
    grid=(N // RB,),
    in_specs=[
        pl.BlockSpec((RB, 2), lambda i: (i, 0)),
        pl.BlockSpec((RB, D_IN), lambda i: (i, 0)),
        pl.BlockSpec((D_IN, D_HID), lambda i: (0, 0)),
    ],
    out_specs=[
        pl.BlockSpec((RB, D_HID // 2), lambda i: (i, 0)),
        pl.BlockSpec((RB, D_HID // 2), lambda i: (i, 0)),
        pl.BlockSpec((RB, 1), lambda i: (i, 0)),
    ],
    out_shape=[
        jax.ShapeDtypeStruct((N, D_HID // 2), jnp.float32),
        jax.ShapeDtypeStruct((N, D_HID // 2), jnp.float32),
        jax.ShapeDtypeStruct((N, 1), jnp.float32),
    ],
)


def _mm2_body(acc0_ref, acc1_ref, hs0_ref, hs1_ref, dinv_ref, b1_ref, w2_ref,
              o_ref, o2_ref):
    dinv = dinv_ref[...]
    pre = jnp.concatenate(
        [acc0_ref[...] + hs0_ref[...], acc1_ref[...] + hs1_ref[...]], axis=1)
    act = jnp.maximum(pre * dinv + b1_ref[...], 0.0)
    h2 = jnp.dot(act, w2_ref[...], preferred_element_type=jnp.float32) * dinv
    o_ref[...] = h2
    o2_ref[...] = h2


_mm2 = pl.pallas_call(
    _mm2_body,
    grid=(N // RB,),
    in_specs=[
        pl.BlockSpec((RB, D_HID // 2), lambda i: (i, 0)),
        pl.BlockSpec((RB, D_HID // 2), lambda i: (i, 0)),
        pl.BlockSpec((RB, D_HID // 2), lambda i: (i, 0)),
        pl.BlockSpec((RB, D_HID // 2), lambda i: (i, 0)),
        pl.BlockSpec((RB, 1), lambda i: (i, 0)),
        pl.BlockSpec((1, D_HID), lambda i: (0, 0)),
        pl.BlockSpec((D_HID, D_OUT), lambda i: (0, 0)),
    ],
    out_specs=[
        pl.BlockSpec((RB, D_OUT), lambda i: (i, 0)),
        pl.BlockSpec((RB, D_OUT), lambda i: (i, 0)),
    ],
    out_shape=[
        jax.ShapeDtypeStruct((N, D_OUT), jnp.float32),
        jax.ShapeDtypeStruct((N, D_OUT), jnp.float32),
    ],
)


def _mm3_body(acc0_ref, acc1_ref, hs2_ref, dinv_ref, b2_ref, out_ref):
    pre = acc0_ref[...] + acc1_ref[...] + hs2_ref[...]
    out_ref[...] = pre * dinv_ref[...] + b2_ref[...]


_mm3 = pl.pallas_call(
    _mm3_body,
    grid=(N // RB,),
    in_specs=[
        pl.BlockSpec((RB, D_OUT), lambda i: (i, 0)),
        pl.BlockSpec((RB, D_OUT), lambda i: (i, 0)),
        pl.BlockSpec((RB, D_OUT), lambda i: (i, 0)),
        pl.BlockSpec((RB, 1), lambda i: (i, 0)),
        pl.BlockSpec((1, D_OUT), lambda i: (0, 0)),
    ],
    out_specs=pl.BlockSpec((RB, D_OUT), lambda i: (i, 0)),
    out_shape=jax.ShapeDtypeStruct((N, D_OUT), jnp.float32),
)


def kernel(x, edge_index, conv1_weight, conv1_bias, conv2_weight, conv2_bias):
    extra = E_PAD - E
    src_p = jnp.concatenate([edge_index[0], jnp.zeros((extra,), jnp.int32)])
    sink = N + (jnp.arange(extra, dtype=jnp.int32) % (NPAD - N))
    dst_p = jnp.concatenate([edge_index[1], sink])
    src2d = src_p.reshape(EROWS, CW)
    dst2d = dst_p.reshape(EROWS, CW)
    deg_parts = _deg(dst_p)
    degp = deg_parts.reshape(NC, NPAD)[:, :N].T
    hs1a, hs1b, dinv = _mm1(degp, x, conv1_weight)
    acc1a, acc1b = _conv_hid(src2d, dst2d, hs1a, hs1b)
    hs2, hs2x = _mm2(acc1a, acc1b, hs1a, hs1b, dinv,
                     conv1_bias.reshape(1, -1), conv2_weight)
    acc2a, acc2b = _conv_out(src_p, dst_p, hs2)
    return _mm3(acc2a, acc2b, hs2, dinv, conv2_bias.reshape(1, -1))


# speedup vs baseline: 1.0495x; 1.0495x over previous
import functools

import jax
import jax.numpy as jnp
from jax import lax
from jax.experimental import pallas as pl
from jax.experimental.pallas import tpu as pltpu
from jax.experimental.pallas import tpu_sc as plsc

N = 10000
E = 320000
D_IN = 128
D_HID = 256
D_OUT = 128

NC = 2
NS = 16
ROWS_PER_TILE = 640
NPAD = NS * ROWS_PER_TILE
CW = 128
SROWS = 16
EROWS = 2560
E_PAD = EROWS * CW
RB = 400


def _sc_mesh():
    return plsc.VectorSubcoreMesh(core_axis_name="c", subcore_axis_name="s")


HR = NPAD // 128
HRPT = 8
DEG_CHUNK = 2048


@functools.partial(
    pl.kernel,
    mesh=_sc_mesh(),
    compiler_params=pltpu.CompilerParams(needs_layout_passes=False),
    out_type=jax.ShapeDtypeStruct((NC, HR, 128), jnp.float32),
    scratch_types=[
        pltpu.VMEM((DEG_CHUNK,), jnp.int32),
        pltpu.VMEM((HR, 128), jnp.float32),
        pltpu.VMEM((HR,), jnp.int32),
        pltpu.VMEM((HRPT, 128), jnp.float32),
        pltpu.VMEM_SHARED((HR, 128), jnp.float32),
    ],
)
def _deg(dst_hbm, out_hbm, idx_v, hist_v, rowid_v, ob_v, acc_sh):
    c = lax.axis_index("c")
    s = lax.axis_index("s")

    def zrow(j, carry):
        def zcol(l, carry2):
            hist_v[j, pl.ds(l * 16, 16)] = jnp.zeros((16,), jnp.float32)
            return carry2
        return lax.fori_loop(0, 128 // 16, zcol, carry)

    lax.fori_loop(0, HR, zrow, 0)
    for j in range(HR // 16):
        rowid_v[pl.ds(j * 16, 16)] = (
            lax.iota(jnp.int32, 16) + jnp.full((16,), j * 16, jnp.int32))

    @pl.when(s == 0)
    def _():
        pltpu.sync_copy(hist_v, acc_sh)
    plsc.subcore_barrier()

    epw = E_PAD // (NC * NS)
    base = (c * NS + s) * epw

    def chunk_step(k, carry):
        pltpu.sync_copy(dst_hbm.at[pl.ds(base + k * DEG_CHUNK, DEG_CHUNK)],
                        idx_v)

        def vstep(j, carry2):
            v = idx_v[pl.ds(j * 16, 16)]
            skey, _ = plsc.sort_key_val(v, v)
            cnt, last = plsc.scan_count(skey)
            row = lax.shift_right_logical(skey, 7)
            col = jnp.bitwise_and(skey, 127)
            plsc.addupdate_scatter(hist_v, [row, col],
                                   cnt.astype(jnp.float32), mask=last)
            return carry2

        return lax.fori_loop(0, DEG_CHUNK // 16, vstep, carry)

    lax.fori_loop(0, epw // DEG_CHUNK, chunk_step, 0)
    pltpu.sync_copy(hist_v, acc_sh.at[rowid_v], add=True)
    plsc.subcore_barrier()

    @pl.when(s < HR // HRPT)
    def _():
        pltpu.sync_copy(acc_sh.at[pl.ds(s * HRPT, HRPT)], ob_v)

        @pl.when(c == 0)
        def _():
            pltpu.sync_copy(ob_v, out_hbm.at[0, pl.ds(s * HRPT, HRPT)])

        @pl.when(c == 1)
        def _():
            pltpu.sync_copy(ob_v, out_hbm.at[1, pl.ds(s * HRPT, HRPT)])


def _make_conv(F, feature_split, pipelined=True):
    zr = 64
    nz = ROWS_PER_TILE // zr
    rpt = EROWS // NS if feature_split else EROWS // (NC * NS)
    nstages = rpt // SROWS

    n_hs = 2 if feature_split else 1

    @functools.partial(
        pl.kernel,
        mesh=_sc_mesh(),
        out_type=[
            jax.ShapeDtypeStruct((NPAD, F), jnp.float32),
            jax.ShapeDtypeStruct((NPAD, F), jnp.float32),
        ],
        scratch_types=[
            pltpu.VMEM((SROWS, CW), jnp.int32),
            pltpu.VMEM((SROWS, CW), jnp.int32),
            pltpu.VMEM((CW, F), jnp.float32),
            pltpu.VMEM((CW, F), jnp.float32),
            pltpu.VMEM((zr, F), jnp.float32),
            pltpu.VMEM((80,), jnp.int32),
            pltpu.VMEM((80,), jnp.int32),
            pltpu.VMEM((80, F), jnp.float32),
            pltpu.SemaphoreType.DMA,
            pltpu.SemaphoreType.DMA,
            pltpu.VMEM_SHARED((NPAD, F), jnp.float32),
        ],
    )
    def conv(src_hbm, dst_hbm, *rest):
        hs_hbms = rest[:n_hs]
        out0_hbm, out1_hbm = rest[n_hs:n_hs + 2]
        (sidx, didx, rb0, rb1, zb_v, si80, di80, r80,
         sm0, sm1, acc_sh) = rest[n_hs + 2:]
        bufs = (rb0, rb1)
        sems = (sm0, sm1)
        NBUF = 2
        c = lax.axis_index("c")
        s = lax.axis_index("s")

        def zrow(j, carry):
            def zcol(l, carry2):
                zb_v[j, pl.ds(l * 16, 16)] = jnp.zeros((16,), jnp.float32)
                return carry2
            return lax.fori_loop(0, F // 16, zcol, carry)

        lax.fori_loop(0, zr, zrow, 0)
        row0 = s * ROWS_PER_TILE
        for t in range(nz):
            pltpu.sync_copy(zb_v, acc_sh.at[pl.ds(row0 + t * zr, zr)])
        plsc.subcore_barrier()

        def pass_edges(hs_hbm, first_stage, stride):
            def stage(oo, carry):
                r0 = (first_stage + oo * stride) * SROWS
                pltpu.sync_copy(src_hbm.at[pl.ds(r0, SROWS)], sidx)
                pltpu.sync_copy(dst_hbm.at[pl.ds(r0, SROWS)], didx)
                for u in range(NBUF - 1):
                    pltpu.async_copy(hs_hbm.at[sidx.at[u]], bufs[u], sems[u])

                def quad(q, c2):
                    j0 = q * NBUF
                    for u in range(NBUF):
                        j = j0 + u
                        nxt = j + NBUF - 1
                        bn = (u + NBUF - 1) % NBUF

                        @pl.when(nxt < SROWS)
                        def _():
                            pltpu.async_copy(
                                hs_hbm.at[sidx.at[nxt]], bufs[bn], sems[bn])

                        pltpu.make_async_copy(
                            hs_hbm.at[pl.ds(0, CW)], bufs[u], sems[u]).wait()
                        pltpu.sync_copy(
                            bufs[u], acc_sh.at[didx.at[j]], add=True)
                    return c2

                return lax.fori_loop(0, SROWS // NBUF, quad, carry)

            lax.fori_loop(0, nstages, stage, 0)

        if not pipelined:
            ept = E_PAD // (NC * NS)

            def step(i, carry):
                off = (c * NS + s) * ept + i * 80
                pltpu.sync_copy(src_hbm.at[pl.ds(off, 80)], si80)
                pltpu.sync_copy(dst_hbm.at[pl.ds(off, 80)], di80)
                pltpu.async_copy(hs_hbms[0].at[si80], r80, sems[0]).wait()
                pltpu.sync_copy(r80, acc_sh.at[di80], add=True)
                return carry

            lax.fori_loop(0, ept // 80, step, 0)
        elif feature_split:
            @pl.when(c == 0)
            def _():
                pass_edges(hs_hbms[0], s, NS)

            @pl.when(c == 1)
            def _():
                pass_edges(hs_hbms[1], s, NS)


        plsc.subcore_barrier()
        for t in range(nz):
            pltpu.sync_copy(acc_sh.at[pl.ds(row0 + t * zr, zr)], zb_v)

            @pl.when(c == 0)
            def _():
                pltpu.sync_copy(zb_v, out0_hbm.at[pl.ds(row0 + t * zr, zr)])

            @pl.when(c == 1)
            def _():
                pltpu.sync_copy(zb_v, out1_hbm.at[pl.ds(row0 + t * zr, zr)])

    return conv


_conv_hid = _make_conv(D_HID // 2, feature_split=True)
_conv_out = _make_conv(D_OUT, feature_split=False, pipelined=False)


def _mm1_body(degp_ref, x_ref, w_ref, hs0_ref, hs1_ref, dinv_ref):
    deg = degp_ref[:, 0] + degp_ref[:, 1] + 1.0
    dinv = lax.rsqrt(deg)
    h = jnp.dot(x_ref[...], w_ref[...], preferred_element_type=jnp.float32)
    hs = h * dinv[:, None]
    hs0_ref[...] = hs[:, : D_HID // 2]
    hs1_ref[...] = hs[:, D_HID // 2:]
    dinv_ref[...] = dinv[:, None]


_mm1 = pl.pallas_call(
    _mm1_body,
    grid=(N // RB,),
    in_specs=[
        pl.BlockSpec((RB, 2), lambda i: (i, 0)),
        pl.BlockSpec((RB, D_IN), lambda i: (i, 0)),
        pl.BlockSpec((D_IN, D_HID), lambda i: (0, 0)),
    ],
    out_specs=[
        pl.BlockSpec((RB, D_HID // 2), lambda i: (i, 0)),
        pl.BlockSpec((RB, D_HID // 2), lambda i: (i, 0)),
        pl.BlockSpec((RB, 1), lambda i: (i, 0)),
    ],
    out_shape=[
        jax.ShapeDtypeStruct((N, D_HID // 2), jnp.float32),
        jax.ShapeDtypeStruct((N, D_HID // 2), jnp.float32),
        jax.ShapeDtypeStruct((N, 1), jnp.float32),
    ],
)


def _mm2_body(acc0_ref, acc1_ref, hs0_ref, hs1_ref, dinv_ref, b1_ref, w2_ref,
              o_ref):
    dinv = dinv_ref[...]
    pre = jnp.concatenate(
        [acc0_ref[...] + hs0_ref[...], acc1_ref[...] + hs1_ref[...]], axis=1)
    act = jnp.maximum(pre * dinv + b1_ref[...], 0.0)
    o_ref[...] = jnp.dot(act, w2_ref[...],
                         preferred_element_type=jnp.float32) * dinv


_mm2 = pl.pallas_call(
    _mm2_body,
    grid=(N // RB,),
    in_specs=[
        pl.BlockSpec((RB, D_HID // 2), lambda i: (i, 0)),
        pl.BlockSpec((RB, D_HID // 2), lambda i: (i, 0)),
        pl.BlockSpec((RB, D_HID // 2), lambda i: (i, 0)),
        pl.BlockSpec((RB, D_HID // 2), lambda i: (i, 0)),
        pl.BlockSpec((RB, 1), lambda i: (i, 0)),
        pl.BlockSpec((1, D_HID), lambda i: (0, 0)),
        pl.BlockSpec((D_HID, D_OUT), lambda i: (0, 0)),
    ],
    out_specs=pl.BlockSpec((RB, D_OUT), lambda i: (i, 0)),
    out_shape=jax.ShapeDtypeStruct((N, D_OUT), jnp.float32),
)


def _mm3_body(acc0_ref, acc1_ref, hs2_ref, dinv_ref, b2_ref, out_ref):
    pre = acc0_ref[...] + acc1_ref[...] + hs2_ref[...]
    out_ref[...] = pre * dinv_ref[...] + b2_ref[...]


_mm3 = pl.pallas_call(
    _mm3_body,
    grid=(N // RB,),
    in_specs=[
        pl.BlockSpec((RB, D_OUT), lambda i: (i, 0)),
        pl.BlockSpec((RB, D_OUT), lambda i: (i, 0)),
        pl.BlockSpec((RB, D_OUT), lambda i: (i, 0)),
        pl.BlockSpec((RB, 1), lambda i: (i, 0)),
        pl.BlockSpec((1, D_OUT), lambda i: (0, 0)),
    ],
    out_specs=pl.BlockSpec((RB, D_OUT), lambda i: (i, 0)),
    out_shape=jax.ShapeDtypeStruct((N, D_OUT), jnp.float32),
)


def kernel(x, edge_index, conv1_weight, conv1_bias, conv2_weight, conv2_bias):
    extra = E_PAD - E
    src_p = jnp.concatenate([edge_index[0], jnp.zeros((extra,), jnp.int32)])
    sink = N + (jnp.arange(extra, dtype=jnp.int32) % (NPAD - N))
    dst_p = jnp.concatenate([edge_index[1], sink])
    src2d = src_p.reshape(EROWS, CW)
    dst2d = dst_p.reshape(EROWS, CW)
    deg_parts = _deg(dst_p)
    degp = deg_parts.reshape(NC, NPAD)[:, :N].T
    hs1a, hs1b, dinv = _mm1(degp, x, conv1_weight)
    acc1a, acc1b = _conv_hid(src2d, dst2d, hs1a, hs1b)
    hs2 = _mm2(acc1a, acc1b, hs1a, hs1b, dinv,
               conv1_bias.reshape(1, -1), conv2_weight)
    acc2a, acc2b = _conv_out(src_p, dst_p, hs2)
    return _mm3(acc2a, acc2b, hs2, dinv, conv2_bias.reshape(1, -1))


# speedup vs baseline: 1.2416x; 1.1830x over previous
import functools

import jax
import jax.numpy as jnp
from jax import lax
from jax.experimental import pallas as pl
from jax.experimental.pallas import tpu as pltpu
from jax.experimental.pallas import tpu_sc as plsc

N = 10000
E = 320000
D_IN = 128
D_HID = 256
D_OUT = 128

NC = 2
NS = 16
ROWS_PER_TILE = 640
NPAD = NS * ROWS_PER_TILE
CHUNK = 80
RB = 400


def _sc_mesh():
    return plsc.VectorSubcoreMesh(core_axis_name="c", subcore_axis_name="s")


HR = NPAD // 128
HRPT = 8
DEG_CHUNK = 2000


@functools.partial(
    pl.kernel,
    mesh=_sc_mesh(),
    compiler_params=pltpu.CompilerParams(needs_layout_passes=False),
    out_type=jax.ShapeDtypeStruct((NC, HR, 128), jnp.float32),
    scratch_types=[
        pltpu.VMEM((DEG_CHUNK,), jnp.int32),
        pltpu.VMEM((HR, 128), jnp.float32),
        pltpu.VMEM((HR,), jnp.int32),
        pltpu.VMEM((HRPT, 128), jnp.float32),
        pltpu.VMEM_SHARED((HR, 128), jnp.float32),
    ],
)
def _deg(dst_hbm, out_hbm, idx_v, hist_v, rowid_v, ob_v, acc_sh):
    c = lax.axis_index("c")
    s = lax.axis_index("s")

    def zrow(j, carry):
        def zcol(l, carry2):
            hist_v[j, pl.ds(l * 16, 16)] = jnp.zeros((16,), jnp.float32)
            return carry2
        return lax.fori_loop(0, 128 // 16, zcol, carry)

    lax.fori_loop(0, HR, zrow, 0)
    for j in range(HR // 16):
        rowid_v[pl.ds(j * 16, 16)] = (
            lax.iota(jnp.int32, 16) + jnp.full((16,), j * 16, jnp.int32))

    @pl.when(s == 0)
    def _():
        pltpu.sync_copy(hist_v, acc_sh)
    plsc.subcore_barrier()

    epw = E // (NC * NS)
    base = (c * NS + s) * epw
    ones16 = jnp.full((16,), 1.0, jnp.float32)

    def chunk_step(k, carry):
        pltpu.sync_copy(dst_hbm.at[pl.ds(base + k * DEG_CHUNK, DEG_CHUNK)],
                        idx_v)

        def vstep(j, carry2):
            v = idx_v[pl.ds(j * 16, 16)]
            row = lax.shift_right_logical(v, 7)
            col = jnp.bitwise_and(v, 127)
            plsc.addupdate_scatter(hist_v, [row, col], ones16)
            return carry2

        return lax.fori_loop(0, DEG_CHUNK // 16, vstep, carry)

    lax.fori_loop(0, epw // DEG_CHUNK, chunk_step, 0)
    pltpu.sync_copy(hist_v, acc_sh.at[rowid_v], add=True)
    plsc.subcore_barrier()

    @pl.when(s < HR // HRPT)
    def _():
        pltpu.sync_copy(acc_sh.at[pl.ds(s * HRPT, HRPT)], ob_v)

        @pl.when(c == 0)
        def _():
            pltpu.sync_copy(ob_v, out_hbm.at[0, pl.ds(s * HRPT, HRPT)])

        @pl.when(c == 1)
        def _():
            pltpu.sync_copy(ob_v, out_hbm.at[1, pl.ds(s * HRPT, HRPT)])


def _make_conv(F2):
    ept = E // NS
    nchunk = ept // CHUNK
    zr = 64
    nz = ROWS_PER_TILE // zr

    @functools.partial(
        pl.kernel,
        mesh=_sc_mesh(),
        out_type=[
            jax.ShapeDtypeStruct((NPAD, F2), jnp.float32),
            jax.ShapeDtypeStruct((NPAD, F2), jnp.float32),
        ],
        scratch_types=[
            pltpu.VMEM((CHUNK,), jnp.int32),
            pltpu.VMEM((CHUNK,), jnp.int32),
            pltpu.VMEM((CHUNK, F2), jnp.float32),
            pltpu.VMEM((zr, F2), jnp.float32),
            pltpu.SemaphoreType.DMA,
            pltpu.VMEM_SHARED((NPAD, F2), jnp.float32),
        ],
    )
    def conv(src_hbm, dst_hbm, hs0_hbm, hs1_hbm, out0_hbm, out1_hbm,
             si_v, di_v, rows_v, zb_v, sem, acc_sh):
        c = lax.axis_index("c")
        s = lax.axis_index("s")

        def zrow(j, carry):
            def zcol(l, carry2):
                zb_v[j, pl.ds(l * 16, 16)] = jnp.zeros((16,), jnp.float32)
                return carry2
            return lax.fori_loop(0, F2 // 16, zcol, carry)

        lax.fori_loop(0, zr, zrow, 0)
        row0 = s * ROWS_PER_TILE
        for t in range(nz):
            pltpu.sync_copy(zb_v, acc_sh.at[pl.ds(row0 + t * zr, zr)])
        plsc.subcore_barrier()

        def pass_edges(hs_hbm):
            base = s * ept

            def step(i, carry):
                off = base + i * CHUNK
                pltpu.sync_copy(src_hbm.at[pl.ds(off, CHUNK)], si_v)
                pltpu.sync_copy(dst_hbm.at[pl.ds(off, CHUNK)], di_v)
                pltpu.async_copy(hs_hbm.at[si_v], rows_v, sem).wait()
                pltpu.sync_copy(rows_v, acc_sh.at[di_v], add=True)
                return carry

            lax.fori_loop(0, nchunk, step, 0)

        @pl.when(c == 0)
        def _():
            pass_edges(hs0_hbm)

        @pl.when(c == 1)
        def _():
            pass_edges(hs1_hbm)

        plsc.subcore_barrier()
        for t in range(nz):
            pltpu.sync_copy(acc_sh.at[pl.ds(row0 + t * zr, zr)], zb_v)

            @pl.when(c == 0)
            def _():
                pltpu.sync_copy(zb_v, out0_hbm.at[pl.ds(row0 + t * zr, zr)])

            @pl.when(c == 1)
            def _():
                pltpu.sync_copy(zb_v, out1_hbm.at[pl.ds(row0 + t * zr, zr)])

    return conv


_conv_hid = _make_conv(D_HID // 2)


def _make_conv_es(F):
    ept = E // (NC * NS)
    nchunk = ept // CHUNK
    zr = 64
    nz = ROWS_PER_TILE // zr

    @functools.partial(
        pl.kernel,
        mesh=_sc_mesh(),
        out_type=[
            jax.ShapeDtypeStruct((NPAD, F), jnp.float32),
            jax.ShapeDtypeStruct((NPAD, F), jnp.float32),
        ],
        scratch_types=[
            pltpu.VMEM((CHUNK,), jnp.int32),
            pltpu.VMEM((CHUNK,), jnp.int32),
            pltpu.VMEM((CHUNK, F), jnp.float32),
            pltpu.VMEM((zr, F), jnp.float32),
            pltpu.SemaphoreType.DMA,
            pltpu.VMEM_SHARED((NPAD, F), jnp.float32),
        ],
    )
    def conv(src_hbm, dst_hbm, hs_hbm, out0_hbm, out1_hbm,
             si_v, di_v, rows_v, zb_v, sem, acc_sh):
        c = lax.axis_index("c")
        s = lax.axis_index("s")

        def zrow(j, carry):
            def zcol(l, carry2):
                zb_v[j, pl.ds(l * 16, 16)] = jnp.zeros((16,), jnp.float32)
                return carry2
            return lax.fori_loop(0, F // 16, zcol, carry)

        lax.fori_loop(0, zr, zrow, 0)
        row0 = s * ROWS_PER_TILE
        for t in range(nz):
            pltpu.sync_copy(zb_v, acc_sh.at[pl.ds(row0 + t * zr, zr)])
        plsc.subcore_barrier()

        base = (c * NS + s) * ept

        def step(i, carry):
            off = base + i * CHUNK
            pltpu.sync_copy(src_hbm.at[pl.ds(off, CHUNK)], si_v)
            pltpu.sync_copy(dst_hbm.at[pl.ds(off, CHUNK)], di_v)
            pltpu.async_copy(hs_hbm.at[si_v], rows_v, sem).wait()
            pltpu.sync_copy(rows_v, acc_sh.at[di_v], add=True)
            return carry

        lax.fori_loop(0, nchunk, step, 0)
        plsc.subcore_barrier()
        for t in range(nz):
            pltpu.sync_copy(acc_sh.at[pl.ds(row0 + t * zr, zr)], zb_v)

            @pl.when(c == 0)
            def _():
                pltpu.sync_copy(zb_v, out0_hbm.at[pl.ds(row0 + t * zr, zr)])

            @pl.when(c == 1)
            def _():
                pltpu.sync_copy(zb_v, out1_hbm.at[pl.ds(row0 + t * zr, zr)])

    return conv


_conv_out = _make_conv_es(D_OUT)


def _mm1_body(degp_ref, x_ref, w_ref, hs0_ref, hs1_ref, dinv_ref):
    deg = degp_ref[:, 0] + degp_ref[:, 1] + 1.0
    dinv = lax.rsqrt(deg)
    h = jnp.dot(x_ref[...], w_ref[...], preferred_element_type=jnp.float32)
    hs = h * dinv[:, None]
    hs0_ref[...] = hs[:, : D_HID // 2]
    hs1_ref[...] = hs[:, D_HID // 2:]
    dinv_ref[...] = dinv[:, None]


_mm1 = pl.pallas_call(
    _mm1_body,
    grid=(N // RB,),
    in_specs=[
        pl.BlockSpec((RB, 2), lambda i: (i, 0)),
        pl.BlockSpec((RB, D_IN), lambda i: (i, 0)),
        pl.BlockSpec((D_IN, D_HID), lambda i: (0, 0)),
    ],
    out_specs=[
        pl.BlockSpec((RB, D_HID // 2), lambda i: (i, 0)),
        pl.BlockSpec((RB, D_HID // 2), lambda i: (i, 0)),
        pl.BlockSpec((RB, 1), lambda i: (i, 0)),
    ],
    out_shape=[
        jax.ShapeDtypeStruct((N, D_HID // 2), jnp.float32),
        jax.ShapeDtypeStruct((N, D_HID // 2), jnp.float32),
        jax.ShapeDtypeStruct((N, 1), jnp.float32),
    ],
)


def _mm2_body(acc0_ref, acc1_ref, hs0_ref, hs1_ref, dinv_ref, b1_ref, w2_ref,
              o_ref):
    dinv = dinv_ref[...]
    pre = jnp.concatenate(
        [acc0_ref[...] + hs0_ref[...], acc1_ref[...] + hs1_ref[...]], axis=1)
    act = jnp.maximum(pre * dinv + b1_ref[...], 0.0)
    o_ref[...] = jnp.dot(act, w2_ref[...],
                         preferred_element_type=jnp.float32) * dinv


_mm2 = pl.pallas_call(
    _mm2_body,
    grid=(N // RB,),
    in_specs=[
        pl.BlockSpec((RB, D_HID // 2), lambda i: (i, 0)),
        pl.BlockSpec((RB, D_HID // 2), lambda i: (i, 0)),
        pl.BlockSpec((RB, D_HID // 2), lambda i: (i, 0)),
        pl.BlockSpec((RB, D_HID // 2), lambda i: (i, 0)),
        pl.BlockSpec((RB, 1), lambda i: (i, 0)),
        pl.BlockSpec((1, D_HID), lambda i: (0, 0)),
        pl.BlockSpec((D_HID, D_OUT), lambda i: (0, 0)),
    ],
    out_specs=pl.BlockSpec((RB, D_OUT), lambda i: (i, 0)),
    out_shape=jax.ShapeDtypeStruct((N, D_OUT), jnp.float32),
)


def _mm3_body(acc0_ref, acc1_ref, hs2_ref, dinv_ref, b2_ref, out_ref):
    pre = acc0_ref[...] + acc1_ref[...] + hs2_ref[...]
    out_ref[...] = pre * dinv_ref[...] + b2_ref[...]


_mm3 = pl.pallas_call(
    _mm3_body,
    grid=(N // RB,),
    in_specs=[
        pl.BlockSpec((RB, D_OUT), lambda i: (i, 0)),
        pl.BlockSpec((RB, D_OUT), lambda i: (i, 0)),
        pl.BlockSpec((RB, D_OUT), lambda i: (i, 0)),
        pl.BlockSpec((RB, 1), lambda i: (i, 0)),
        pl.BlockSpec((1, D_OUT), lambda i: (0, 0)),
    ],
    out_specs=pl.BlockSpec((RB, D_OUT), lambda i: (i, 0)),
    out_shape=jax.ShapeDtypeStruct((N, D_OUT), jnp.float32),
)


def kernel(x, edge_index, conv1_weight, conv1_bias, conv2_weight, conv2_bias):
    src = edge_index[0]
    dst = edge_index[1]
    deg_parts = _deg(dst)
    degp = deg_parts.reshape(NC, NPAD)[:, :N].T
    hs1a, hs1b, dinv = _mm1(degp, x, conv1_weight)
    acc1a, acc1b = _conv_hid(src, dst, hs1a, hs1b)
    hs2 = _mm2(acc1a, acc1b, hs1a, hs1b, dinv,
               conv1_bias.reshape(1, -1), conv2_weight)
    acc2a, acc2b = _conv_out(src, dst, hs2)
    return _mm3(acc2a, acc2b, hs2, dinv, conv2_bias.reshape(1, -1))
